# baseline (speedup 1.0000x reference)
import functools

import jax
import jax.numpy as jnp
from jax import lax
from jax.experimental import pallas as pl
from jax.experimental.pallas import tpu as pltpu
from jax.experimental.pallas import tpu_sc as plsc

REG_LAMBDA = 0.01
LANES = 128
SUB = 3
N_CHUNKS = 2


def _gather_body(chunks, h_i, p_i, n_i, r_i, etab, rtab,
                 h_o, p_o, n_o, r_o, idx_v, rows_v, gsem, osem0, osem1):
    info = plsc.get_sparse_core_info()
    nc = info.num_cores
    wid = lax.axis_index("s") * nc + lax.axis_index("c")
    osems = (osem0, osem1)
    state = {"si": 0, "pending": [None, None]}

    def run(idx_hbm, tab_hbm, out_hbm):
        pltpu.sync_copy(idx_hbm.at[wid], idx_v)
        rows_per_w = chunks * LANES
        for s in range(0, chunks, SUB):
            k = min(SUB, chunks - s)
            buf = state["si"] % 2
            state["si"] += 1
            if state["pending"][buf] is not None:
                state["pending"][buf].wait()
            copies = []
            for j in range(k):
                copies.append(pltpu.async_copy(
                    tab_hbm.at[idx_v.at[s + j]],
                    rows_v.at[buf, pl.ds(j * LANES, LANES)], gsem))
            for c in copies:
                c.wait()
            state["pending"][buf] = pltpu.async_copy(
                rows_v.at[buf, pl.ds(0, k * LANES)],
                out_hbm.at[pl.ds(wid * rows_per_w + s * LANES, k * LANES)],
                osems[buf])

    run(h_i, etab, h_o)
    run(p_i, etab, p_o)
    run(n_i, etab, n_o)
    run(r_i, rtab, r_o)
    for p in state["pending"]:
        if p is not None:
            p.wait()


def _sc_gather(h_i, p_i, n_i, r_i, entity_table, relation_table):
    nw, chunks, _ = h_i.shape
    bc = nw * chunks * LANES
    d = entity_table.shape[1]
    mesh = plsc.VectorSubcoreMesh(core_axis_name="c", subcore_axis_name="s")
    row_t = jax.ShapeDtypeStruct((bc, d), jnp.float32)
    kern = functools.partial(
        pl.kernel,
        mesh=mesh,
        out_type=[row_t, row_t, row_t, row_t],
        scratch_types=[
            pltpu.VMEM((chunks, LANES), jnp.int32),
            pltpu.VMEM((2, SUB * LANES, d), jnp.float32),
            pltpu.SemaphoreType.DMA,
            pltpu.SemaphoreType.DMA,
            pltpu.SemaphoreType.DMA,
        ],
    )(functools.partial(_gather_body, chunks))
    return kern(h_i, p_i, n_i, r_i, entity_table, relation_table)


def _loss_body(nb, gh, gp, gn, gr, w_ref, out_ref, acc_ref):
    i = pl.program_id(0)

    @pl.when(i == 0)
    def _():
        acc_ref[0] = 0.0
        acc_ref[1] = 0.0

    w = w_ref[...]
    he = jnp.abs(jnp.dot(gh[...], w, preferred_element_type=jnp.float32))
    pe = jnp.abs(jnp.dot(gp[...], w, preferred_element_type=jnp.float32))
    ne = jnp.abs(jnp.dot(gn[...], w, preferred_element_type=jnp.float32))
    re = jnp.abs(gr[...])

    base = he + re
    dpos = base - pe
    dneg = base - ne
    pos_s = 0.5 * jnp.sum(dpos * dpos, axis=1, keepdims=True)
    neg_s = 0.5 * jnp.sum(dneg * dneg, axis=1, keepdims=True)
    x = neg_s - pos_s
    logsig = jnp.minimum(x, 0.0) - jnp.log1p(jnp.exp(-jnp.abs(x)))
    sq = (jnp.sum(he * he) + jnp.sum(re * re)
          + jnp.sum(pe * pe) + jnp.sum(ne * ne))
    acc_ref[0] += jnp.sum(logsig)
    acc_ref[1] += sq

    @pl.when(i == nb - 1)
    def _():
        out_ref[0, 0] = acc_ref[0]
        out_ref[0, 1] = acc_ref[1]


def _tc_partial(gh, gp, gn, gr, W):
    bc, d = gh.shape
    bsz = 2048
    nb = bc // bsz
    spec = pl.BlockSpec((bsz, d), lambda i: (i, 0))
    return pl.pallas_call(
        functools.partial(_loss_body, nb),
        grid=(nb,),
        in_specs=[spec, spec, spec, spec,
                  pl.BlockSpec((d, d), lambda i: (0, 0))],
        out_specs=pl.BlockSpec(memory_space=pltpu.SMEM),
        out_shape=jax.ShapeDtypeStruct((1, 2), jnp.float32),
        scratch_shapes=[pltpu.SMEM((2,), jnp.float32)],
    )(gh, gp, gn, gr, W)


def kernel(h, r, pos_t, neg_t, entity_table, relation_table, W):
    b = h.shape[0]
    info = plsc.get_sparse_core_info()
    nw = info.num_cores * info.num_subcores
    bc = b // N_CHUNKS
    chunks = bc // (nw * LANES)

    def shape_idx(x):
        return x.reshape(N_CHUNKS, nw, chunks, LANES).astype(jnp.int32)

    hi, ri, pi, ni = (shape_idx(x) for x in (h, r, pos_t, neg_t))

    partials = []
    for c in range(N_CHUNKS):
        gh, gp, gn, gr = _sc_gather(hi[c], pi[c], ni[c], ri[c],
                                    entity_table, relation_table)
        partials.append(_tc_partial(gh, gp, gn, gr, W))
    acc = partials[0]
    for p in partials[1:]:
        acc = acc + p
    b_total = jnp.float32(b)
    return (-acc[0, 0] / b_total
            + REG_LAMBDA * acc[0, 1] / (2.0 * b_total))


# speedup vs baseline: 1.0181x; 1.0181x over previous
import functools

import jax
import jax.numpy as jnp
from jax import lax
from jax.experimental import pallas as pl
from jax.experimental.pallas import tpu as pltpu
from jax.experimental.pallas import tpu_sc as plsc

REG_LAMBDA = 0.01
LANES = 128
SUB = 3
N_CHUNKS = 2


def _gather_body(chunks, woff, h_i, p_i, n_i, r_i, etab, rtab,
                 h_o, p_o, n_o, r_o, idx_v, rows_v, gsem, osem0, osem1):
    info = plsc.get_sparse_core_info()
    nc = info.num_cores
    wid = lax.axis_index("s") * nc + lax.axis_index("c")
    osems = (osem0, osem1)
    state = {"si": 0, "pending": [None, None]}

    def run(idx_hbm, tab_hbm, out_hbm):
        pltpu.sync_copy(idx_hbm.at[woff + wid], idx_v)
        rows_per_w = chunks * LANES
        for s in range(0, chunks, SUB):
            k = min(SUB, chunks - s)
            buf = state["si"] % 2
            state["si"] += 1
            if state["pending"][buf] is not None:
                state["pending"][buf].wait()
            copies = []
            for j in range(k):
                copies.append(pltpu.async_copy(
                    tab_hbm.at[idx_v.at[s + j]],
                    rows_v.at[buf, pl.ds(j * LANES, LANES)], gsem))
            for c in copies:
                c.wait()
            state["pending"][buf] = pltpu.async_copy(
                rows_v.at[buf, pl.ds(0, k * LANES)],
                out_hbm.at[pl.ds(wid * rows_per_w + s * LANES, k * LANES)],
                osems[buf])

    run(h_i, etab, h_o)
    run(p_i, etab, p_o)
    run(n_i, etab, n_o)
    run(r_i, rtab, r_o)
    for p in state["pending"]:
        if p is not None:
            p.wait()


def _sc_gather(cidx, h_i, p_i, n_i, r_i, entity_table, relation_table):
    nrows, chunks, _ = h_i.shape
    nw = nrows // N_CHUNKS
    bc = nw * chunks * LANES
    d = entity_table.shape[1]
    mesh = plsc.VectorSubcoreMesh(core_axis_name="c", subcore_axis_name="s")
    row_t = jax.ShapeDtypeStruct((bc, d), jnp.float32)
    kern = functools.partial(
        pl.kernel,
        mesh=mesh,
        out_type=[row_t, row_t, row_t, row_t],
        scratch_types=[
            pltpu.VMEM((chunks, LANES), jnp.int32),
            pltpu.VMEM((2, SUB * LANES, d), jnp.float32),
            pltpu.SemaphoreType.DMA,
            pltpu.SemaphoreType.DMA,
            pltpu.SemaphoreType.DMA,
        ],
    )(functools.partial(_gather_body, chunks, cidx * nw))
    return kern(h_i, p_i, n_i, r_i, entity_table, relation_table)


def _loss_body(nb, gh, gp, gn, gr, w_ref, out_ref, acc_ref):
    i = pl.program_id(0)

    @pl.when(i == 0)
    def _():
        acc_ref[0] = 0.0
        acc_ref[1] = 0.0

    w = w_ref[...]
    he = jnp.abs(jnp.dot(gh[...], w, preferred_element_type=jnp.float32))
    pe = jnp.abs(jnp.dot(gp[...], w, preferred_element_type=jnp.float32))
    ne = jnp.abs(jnp.dot(gn[...], w, preferred_element_type=jnp.float32))
    re = jnp.abs(gr[...])

    base = he + re
    dpos = base - pe
    dneg = base - ne
    pos_s = 0.5 * jnp.sum(dpos * dpos, axis=1, keepdims=True)
    neg_s = 0.5 * jnp.sum(dneg * dneg, axis=1, keepdims=True)
    x = neg_s - pos_s
    logsig = jnp.minimum(x, 0.0) - jnp.log1p(jnp.exp(-jnp.abs(x)))
    sq = (jnp.sum(he * he) + jnp.sum(re * re)
          + jnp.sum(pe * pe) + jnp.sum(ne * ne))
    acc_ref[0] += jnp.sum(logsig)
    acc_ref[1] += sq

    @pl.when(i == nb - 1)
    def _():
        out_ref[0, 0] = acc_ref[0]
        out_ref[0, 1] = acc_ref[1]


def _tc_partial(gh, gp, gn, gr, W):
    bc, d = gh.shape
    bsz = 2048
    nb = bc // bsz
    spec = pl.BlockSpec((bsz, d), lambda i: (i, 0))
    return pl.pallas_call(
        functools.partial(_loss_body, nb),
        grid=(nb,),
        in_specs=[spec, spec, spec, spec,
                  pl.BlockSpec((d, d), lambda i: (0, 0))],
        out_specs=pl.BlockSpec(memory_space=pltpu.SMEM),
        out_shape=jax.ShapeDtypeStruct((1, 2), jnp.float32),
        scratch_shapes=[pltpu.SMEM((2,), jnp.float32)],
    )(gh, gp, gn, gr, W)


def kernel(h, r, pos_t, neg_t, entity_table, relation_table, W):
    b = h.shape[0]
    info = plsc.get_sparse_core_info()
    nw = info.num_cores * info.num_subcores
    bc = b // N_CHUNKS
    chunks = bc // (nw * LANES)

    def shape_idx(x):
        return x.reshape(N_CHUNKS * nw, chunks, LANES).astype(jnp.int32)

    hi, ri, pi, ni = (shape_idx(x) for x in (h, r, pos_t, neg_t))

    partials = []
    for c in range(N_CHUNKS):
        gh, gp, gn, gr = _sc_gather(c, hi, pi, ni, ri,
                                    entity_table, relation_table)
        partials.append(_tc_partial(gh, gp, gn, gr, W))
    acc = partials[0]
    for p in partials[1:]:
        acc = acc + p
    b_total = jnp.float32(b)
    return (-acc[0, 0] / b_total
            + REG_LAMBDA * acc[0, 1] / (2.0 * b_total))


# speedup vs baseline: 1.0973x; 1.0778x over previous
import functools

import jax
import jax.numpy as jnp
from jax import lax
from jax.experimental import pallas as pl
from jax.experimental.pallas import tpu as pltpu
from jax.experimental.pallas import tpu_sc as plsc

REG_LAMBDA = 0.01
LANES = 128
SUB = 3
N_CHUNKS = 2


def _gather_body(chunks, woff, h_i, p_i, n_i, r_i, etab, rtab,
                 h_o, p_o, n_o, r_o, idx_v, rows_v, rel_sp,
                 gsem, osem0, osem1, ssem):
    info = plsc.get_sparse_core_info()
    nc = info.num_cores
    sid = lax.axis_index("s")
    wid = sid * nc + lax.axis_index("c")
    osems = (osem0, osem1)
    state = {"si": 0, "pending": [None, None]}

    @pl.when(sid == 0)
    def _():
        pltpu.async_copy(rtab, rel_sp, ssem)

    def run(idx_hbm, tab_hbm, out_hbm):
        pltpu.sync_copy(idx_hbm.at[woff + wid], idx_v)
        rows_per_w = chunks * LANES
        for s in range(0, chunks, SUB):
            k = min(SUB, chunks - s)
            buf = state["si"] % 2
            state["si"] += 1
            if state["pending"][buf] is not None:
                state["pending"][buf].wait()
            copies = []
            for j in range(k):
                copies.append(pltpu.async_copy(
                    tab_hbm.at[idx_v.at[s + j]],
                    rows_v.at[buf, pl.ds(j * LANES, LANES)], gsem))
            for c in copies:
                c.wait()
            state["pending"][buf] = pltpu.async_copy(
                rows_v.at[buf, pl.ds(0, k * LANES)],
                out_hbm.at[pl.ds(wid * rows_per_w + s * LANES, k * LANES)],
                osems[buf])

    run(h_i, etab, h_o)
    run(p_i, etab, p_o)
    run(n_i, etab, n_o)

    @pl.when(sid == 0)
    def _():
        pltpu.make_async_copy(rtab, rel_sp, ssem).wait()

    plsc.subcore_barrier()
    run(r_i, rel_sp, r_o)
    for p in state["pending"]:
        if p is not None:
            p.wait()


def _sc_gather(cidx, h_i, p_i, n_i, r_i, entity_table, relation_table):
    nrows, chunks, _ = h_i.shape
    nw = nrows // N_CHUNKS
    bc = nw * chunks * LANES
    d = entity_table.shape[1]
    mesh = plsc.VectorSubcoreMesh(core_axis_name="c", subcore_axis_name="s")
    row_t = jax.ShapeDtypeStruct((bc, d), jnp.float32)
    kern = functools.partial(
        pl.kernel,
        mesh=mesh,
        out_type=[row_t, row_t, row_t, row_t],
        scratch_types=[
            pltpu.VMEM((chunks, LANES), jnp.int32),
            pltpu.VMEM((2, SUB * LANES, d), jnp.float32),
            pltpu.VMEM_SHARED(relation_table.shape, jnp.float32),
            pltpu.SemaphoreType.DMA,
            pltpu.SemaphoreType.DMA,
            pltpu.SemaphoreType.DMA,
            pltpu.SemaphoreType.DMA,
        ],
    )(functools.partial(_gather_body, chunks, cidx * nw))
    return kern(h_i, p_i, n_i, r_i, entity_table, relation_table)


def _loss_body(nb, gh, gp, gn, gr, w_ref, out_ref, acc_ref):
    i = pl.program_id(0)

    @pl.when(i == 0)
    def _():
        acc_ref[0] = 0.0
        acc_ref[1] = 0.0

    w = w_ref[...]
    he = jnp.abs(jnp.dot(gh[...], w, preferred_element_type=jnp.float32))
    pe = jnp.abs(jnp.dot(gp[...], w, preferred_element_type=jnp.float32))
    ne = jnp.abs(jnp.dot(gn[...], w, preferred_element_type=jnp.float32))
    re = jnp.abs(gr[...])

    base = he + re
    dpos = base - pe
    dneg = base - ne
    pos_s = 0.5 * jnp.sum(dpos * dpos, axis=1, keepdims=True)
    neg_s = 0.5 * jnp.sum(dneg * dneg, axis=1, keepdims=True)
    x = neg_s - pos_s
    logsig = jnp.minimum(x, 0.0) - jnp.log1p(jnp.exp(-jnp.abs(x)))
    sq = (jnp.sum(he * he) + jnp.sum(re * re)
          + jnp.sum(pe * pe) + jnp.sum(ne * ne))
    acc_ref[0] += jnp.sum(logsig)
    acc_ref[1] += sq

    @pl.when(i == nb - 1)
    def _():
        out_ref[0, 0] = acc_ref[0]
        out_ref[0, 1] = acc_ref[1]


def _tc_partial(gh, gp, gn, gr, W):
    bc, d = gh.shape
    bsz = 2048
    nb = bc // bsz
    spec = pl.BlockSpec((bsz, d), lambda i: (i, 0))
    return pl.pallas_call(
        functools.partial(_loss_body, nb),
        grid=(nb,),
        in_specs=[spec, spec, spec, spec,
                  pl.BlockSpec((d, d), lambda i: (0, 0))],
        out_specs=pl.BlockSpec(memory_space=pltpu.SMEM),
        out_shape=jax.ShapeDtypeStruct((1, 2), jnp.float32),
        scratch_shapes=[pltpu.SMEM((2,), jnp.float32)],
    )(gh, gp, gn, gr, W)


def kernel(h, r, pos_t, neg_t, entity_table, relation_table, W):
    b = h.shape[0]
    info = plsc.get_sparse_core_info()
    nw = info.num_cores * info.num_subcores
    bc = b // N_CHUNKS
    chunks = bc // (nw * LANES)

    def shape_idx(x):
        return x.reshape(N_CHUNKS * nw, chunks, LANES).astype(jnp.int32)

    hi, ri, pi, ni = (shape_idx(x) for x in (h, r, pos_t, neg_t))

    partials = []
    for c in range(N_CHUNKS):
        gh, gp, gn, gr = _sc_gather(c, hi, pi, ni, ri,
                                    entity_table, relation_table)
        partials.append(_tc_partial(gh, gp, gn, gr, W))
    acc = partials[0]
    for p in partials[1:]:
        acc = acc + p
    b_total = jnp.float32(b)
    return (-acc[0, 0] / b_total
            + REG_LAMBDA * acc[0, 1] / (2.0 * b_total))
